# baseline (speedup 1.0000x reference)
import functools

import jax
import jax.numpy as jnp
from jax import lax
from jax.experimental import pallas as pl
from jax.experimental.pallas import tpu as pltpu
from jax.experimental.pallas import tpu_sc as plsc

NU = 25000
NI = 25000
N = NU + NI
D = 64
E2 = 800000
EPT = 819200
W = 128
NWIN = 400
SLICE = NWIN * W
CH = 12544
CHA = 12672
GARB = CH
SUBROWS = 792
ZCH = 198
NCHUNK = 4

_mesh = plsc.VectorSubcoreMesh(core_axis_name="c", subcore_axis_name="s")
_sc_params = pltpu.CompilerParams(use_tc_tiling_on_sc=False)


@functools.partial(
    pl.kernel,
    out_type=jax.ShapeDtypeStruct((NCHUNK * CHA,), jnp.float32),
    mesh=_mesh,
    scratch_types=[
        pltpu.VMEM((W,), jnp.int32),
        pltpu.VMEM((W,), jnp.int32),
        pltpu.VMEM((W,), jnp.float32),
        pltpu.VMEM((SUBROWS,), jnp.float32),
        pltpu.VMEM((SUBROWS,), jnp.float32),
        pltpu.VMEM_SHARED((CHA,), jnp.float32),
    ],
    compiler_params=_sc_params,
)
def _deg_pass(cols_hbm, zeros_hbm, out_hbm, colbuf, dstbuf, onesbuf, stage,
              stage2, acc):
    c = lax.axis_index("c")
    s = lax.axis_index("s")
    iota = lax.iota(jnp.int32, 16)
    one16 = jnp.full((16,), 1.0, jnp.float32)
    for j in range(W // 16):
        onesbuf[pl.ds(j * 16, 16)] = one16
    pltpu.sync_copy(zeros_hbm.at[pl.ds(0, SUBROWS)], stage)

    def eoff(w):
        return pl.multiple_of(s * SLICE + w * W, 8)

    def chunk(p, carry):
        k = 2 * c + p
        base = k * CH
        pltpu.sync_copy(stage, acc.at[pl.ds(s * SUBROWS, SUBROWS)])
        plsc.subcore_barrier()

        def win(w, carry2):
            pltpu.sync_copy(cols_hbm.at[pl.ds(eoff(w), W)], colbuf)
            for j in range(W // 16):
                col16 = colbuf[pl.ds(j * 16, 16)]
                local = col16 - base
                inb = (local >= 0) & (local < CH)
                garb = (GARB + (j % 4) * 16) + iota
                dstbuf[pl.ds(j * 16, 16)] = jnp.where(inb, local, garb)
            pltpu.sync_copy(onesbuf, acc.at[dstbuf], add=True)
            return carry2

        lax.fori_loop(0, NWIN, win, 0)
        plsc.subcore_barrier()
        pltpu.sync_copy(acc.at[pl.ds(s * SUBROWS, SUBROWS)], stage2)
        pltpu.sync_copy(stage2,
                        out_hbm.at[pl.ds(k * CHA + s * SUBROWS, SUBROWS)])
        plsc.subcore_barrier()
        return carry

    lax.fori_loop(0, 2, chunk, 0)


@functools.partial(
    pl.kernel,
    out_type=jax.ShapeDtypeStruct((NCHUNK * CHA, D), jnp.float32),
    mesh=_mesh,
    scratch_types=[
        pltpu.VMEM((2, W), jnp.int32),
        pltpu.VMEM((2, W), jnp.int32),
        pltpu.VMEM((2, W), jnp.int32),
        pltpu.VMEM((2, W, D), jnp.float32),
        pltpu.VMEM((ZCH, D), jnp.float32),
        pltpu.VMEM((ZCH, D), jnp.float32),
        pltpu.VMEM_SHARED((CHA, D), jnp.float32),
        pltpu.SemaphoreType.DMA,
    ],
    compiler_params=_sc_params,
)
def _edge_pass(rows_hbm, cols_hbm, zs_hbm, zeros_hbm, out_hbm,
               row2, col2, dst2, val2, zbuf, iobuf, acc, g0):
    c = lax.axis_index("c")
    s = lax.axis_index("s")
    iota = lax.iota(jnp.int32, 16)
    pltpu.sync_copy(zeros_hbm.at[pl.ds(0, ZCH)], zbuf)

    def eoff(w):
        return pl.multiple_of(s * SLICE + w * W, 8)

    def load_idx(w, sl):
        pltpu.sync_copy(cols_hbm.at[pl.ds(eoff(w), W)], col2.at[sl])
        pltpu.sync_copy(rows_hbm.at[pl.ds(eoff(w), W)], row2.at[sl])

    def chunk(p, carry):
        k = 2 * c + p
        base = k * CH
        for q in range(SUBROWS // ZCH):
            pltpu.sync_copy(zbuf, acc.at[pl.ds(s * SUBROWS + q * ZCH, ZCH)])
        plsc.subcore_barrier()

        def dstc(sl):
            for j in range(W // 16):
                col16 = col2[sl, pl.ds(j * 16, 16)]
                local = col16 - base
                inb = (local >= 0) & (local < CH)
                garb = (GARB + (j % 4) * 16) + iota
                dst2[sl, pl.ds(j * 16, 16)] = jnp.where(inb, local, garb)

        load_idx(0, 0)
        dstc(0)

        def body(w, carry2):
            sl = w & 1
            osl = 1 - sl

            @pl.when(w < NWIN)
            def _():
                pltpu.async_copy(zs_hbm.at[row2.at[sl]], val2.at[sl], g0)

            @pl.when(w > 0)
            def _():
                pltpu.sync_copy(val2.at[osl], acc.at[dst2.at[osl]], add=True)

            @pl.when(w < NWIN)
            def _():
                load_idx(jnp.minimum(w + 1, NWIN - 1), osl)
                dstc(osl)
                pltpu.make_async_copy(zs_hbm.at[row2.at[sl]], val2.at[sl],
                                      g0).wait()

            return carry2

        lax.fori_loop(0, NWIN + 1, body, 0)
        plsc.subcore_barrier()
        for q in range(SUBROWS // ZCH):
            r0 = s * SUBROWS + q * ZCH
            pltpu.sync_copy(acc.at[pl.ds(r0, ZCH)], iobuf)
            pltpu.sync_copy(iobuf, out_hbm.at[pl.ds(k * CHA + r0, ZCH)])
        plsc.subcore_barrier()
        return carry

    lax.fori_loop(0, 2, chunk, 0)


def _matTdot(x, w):
    return lax.dot_general(x, w, (((1,), (1,)), ((), ())),
                           preferred_element_type=jnp.float32)


def _proj(feat, Wm, bm):
    K = feat.shape[1]

    def body(x_ref, w_ref, b_ref, o_ref):
        o_ref[...] = _matTdot(x_ref[...], w_ref[...]) + b_ref[...]

    return pl.pallas_call(
        body,
        grid=(25,),
        in_specs=[pl.BlockSpec((1000, K), lambda i: (i, 0)),
                  pl.BlockSpec((D, K), lambda i: (0, 0)),
                  pl.BlockSpec((1, D), lambda i: (0, 0))],
        out_specs=pl.BlockSpec((1000, D), lambda i: (i, 0)),
        out_shape=jax.ShapeDtypeStruct((NU, D), jnp.float32),
    )(feat, Wm, bm.reshape(1, D))


_RB = 2000


def _first(X0, Wm, deg2):
    def body(x_ref, w_ref, d_ref, o_ref):
        dinv = lax.rsqrt(d_ref[...] + 1.0)
        o_ref[...] = _matTdot(x_ref[...], w_ref[...]) * dinv

    return pl.pallas_call(
        body,
        grid=(N // _RB,),
        in_specs=[pl.BlockSpec((_RB, D), lambda i: (i, 0)),
                  pl.BlockSpec((D, D), lambda i: (0, 0)),
                  pl.BlockSpec((_RB, 1), lambda i: (i, 0))],
        out_specs=pl.BlockSpec((_RB, D), lambda i: (i, 0)),
        out_shape=jax.ShapeDtypeStruct((N, D), jnp.float32),
    )(X0, Wm, deg2)


def _mid(Se, Zs, deg2, bm, Wm):
    def body(se_ref, zs_ref, d_ref, b_ref, w_ref, o_ref):
        dinv = lax.rsqrt(d_ref[...] + 1.0)
        x = dinv * (se_ref[...] + zs_ref[...]) + b_ref[...]
        o_ref[...] = _matTdot(x, w_ref[...]) * dinv

    return pl.pallas_call(
        body,
        grid=(N // _RB,),
        in_specs=[pl.BlockSpec((_RB, D), lambda i: (i, 0)),
                  pl.BlockSpec((_RB, D), lambda i: (i, 0)),
                  pl.BlockSpec((_RB, 1), lambda i: (i, 0)),
                  pl.BlockSpec((1, D), lambda i: (0, 0)),
                  pl.BlockSpec((D, D), lambda i: (0, 0))],
        out_specs=pl.BlockSpec((_RB, D), lambda i: (i, 0)),
        out_shape=jax.ShapeDtypeStruct((N, D), jnp.float32),
    )(Se, Zs, deg2, bm.reshape(1, D), Wm)


def _fin(X0, Se1, Zs1, Se2, Zs2, deg2, b1m, b2m):
    def body(x0_ref, se1_ref, zs1_ref, se2_ref, zs2_ref, d_ref, b1_ref,
             b2_ref, o_ref):
        dinv = lax.rsqrt(d_ref[...] + 1.0)
        x1 = dinv * (se1_ref[...] + zs1_ref[...]) + b1_ref[...]
        x2 = dinv * (se2_ref[...] + zs2_ref[...]) + b2_ref[...]
        o_ref[...] = (x0_ref[...] + x1 + x2) * (1.0 / 3.0)

    rb = pl.BlockSpec((_RB, D), lambda i: (i, 0))
    return pl.pallas_call(
        body,
        grid=(N // _RB,),
        in_specs=[rb, rb, rb, rb, rb,
                  pl.BlockSpec((_RB, 1), lambda i: (i, 0)),
                  pl.BlockSpec((1, D), lambda i: (0, 0)),
                  pl.BlockSpec((1, D), lambda i: (0, 0))],
        out_specs=rb,
        out_shape=jax.ShapeDtypeStruct((N, D), jnp.float32),
    )(X0, Se1, Zs1, Se2, Zs2, deg2, b1m.reshape(1, D), b2m.reshape(1, D))


def _unpad(a2):
    parts = [a2[k * CHA:k * CHA + CH] for k in range(NCHUNK)]
    return jnp.concatenate(parts, axis=0)[:N]


def kernel(edge_index, v_feat, t_feat, user_emb, item_emb, user_emb_v,
           user_emb_t, Wv, bv, Wt, bt, W1, b1, W2, b2):
    ei = edge_index.astype(jnp.int32)
    src = jnp.concatenate([ei[:, 0], ei[:, 1]])
    dst = jnp.concatenate([ei[:, 1], ei[:, 0]])
    npad = EPT - E2
    src_p = jnp.concatenate([src, jnp.arange(npad, dtype=jnp.int32) % N])
    dst_p = jnp.concatenate([dst, jnp.full((npad,), -1, jnp.int32)])

    zeros1 = jnp.zeros((SUBROWS,), jnp.float32)
    zeros2 = jnp.zeros((ZCH, D), jnp.float32)

    deg_p = _deg_pass(dst_p, zeros1)
    deg2 = _unpad(deg_p).reshape(N, 1)

    v_emb = _proj(v_feat, Wv, bv)
    t_emb = _proj(t_feat, Wt, bt)

    def propagate(X0):
        Zs1 = _first(X0, W1, deg2)
        Se1 = _unpad(_edge_pass(src_p, dst_p, Zs1, zeros2))
        Zs2 = _mid(Se1, Zs1, deg2, b1, W2)
        Se2 = _unpad(_edge_pass(src_p, dst_p, Zs2, zeros2))
        M = _fin(X0, Se1, Zs1, Se2, Zs2, deg2, b1, b2)
        return M[:NU], M[NU:]

    u_g, i_g = propagate(jnp.concatenate([user_emb, item_emb], axis=0))
    u_v, i_v = propagate(jnp.concatenate([user_emb_v, v_emb], axis=0))
    u_t, i_t = propagate(jnp.concatenate([user_emb_t, t_emb], axis=0))
    return (u_g, i_g, u_v, i_v, u_t, i_t)


# speedup vs baseline: 1.4590x; 1.4590x over previous
import functools

import jax
import jax.numpy as jnp
from jax import lax
from jax.experimental import pallas as pl
from jax.experimental.pallas import tpu as pltpu
from jax.experimental.pallas import tpu_sc as plsc

NU = 25000
NI = 25000
N = NU + NI
D = 64
E2 = 800000
EPT = 819200
W = 128
NWIN = 400
SLICE = NWIN * W
NP = 25088
HALF = 25000
GARB = 25024
SUBROWS = 1568
NP2 = 50176
GARB2 = 50000
SUB2 = 3136
ZCH = 448
DH = 32

_mesh = plsc.VectorSubcoreMesh(core_axis_name="c", subcore_axis_name="s")
_sc_params = pltpu.CompilerParams(use_tc_tiling_on_sc=False)


@functools.partial(
    pl.kernel,
    out_type=jax.ShapeDtypeStruct((2 * NP,), jnp.float32),
    mesh=_mesh,
    scratch_types=[
        pltpu.VMEM((W,), jnp.int32),
        pltpu.VMEM((W,), jnp.int32),
        pltpu.VMEM((W,), jnp.float32),
        pltpu.VMEM((SUBROWS,), jnp.float32),
        pltpu.VMEM_SHARED((NP,), jnp.float32),
    ],
    compiler_params=_sc_params,
)
def _deg_pass(cols_hbm, zeros_hbm, out_hbm, colbuf, dstbuf, onesbuf, stage,
              acc):
    c = lax.axis_index("c")
    s = lax.axis_index("s")
    base = c * HALF
    iota = lax.iota(jnp.int32, 16)
    one16 = jnp.full((16,), 1.0, jnp.float32)
    for j in range(W // 16):
        onesbuf[pl.ds(j * 16, 16)] = one16
    pltpu.sync_copy(zeros_hbm.at[pl.ds(0, SUBROWS)], stage)
    pltpu.sync_copy(stage, acc.at[pl.ds(s * SUBROWS, SUBROWS)])
    plsc.subcore_barrier()

    def win(w, carry):
        eoff = pl.multiple_of(s * SLICE + w * W, 8)
        pltpu.sync_copy(cols_hbm.at[pl.ds(eoff, W)], colbuf)
        for j in range(W // 16):
            col16 = colbuf[pl.ds(j * 16, 16)]
            local = col16 - base
            inb = (local >= 0) & (local < HALF)
            garb = (GARB + (j % 4) * 16) + iota
            dstbuf[pl.ds(j * 16, 16)] = jnp.where(inb, local, garb)
        pltpu.sync_copy(onesbuf, acc.at[dstbuf], add=True)
        return carry

    lax.fori_loop(0, NWIN, win, 0)
    plsc.subcore_barrier()
    pltpu.sync_copy(acc.at[pl.ds(s * SUBROWS, SUBROWS)], stage)
    pltpu.sync_copy(stage, out_hbm.at[pl.ds(c * NP + s * SUBROWS, SUBROWS)])


@functools.partial(
    pl.kernel,
    out_type=jax.ShapeDtypeStruct((2 * NP2, DH), jnp.float32),
    mesh=_mesh,
    scratch_types=[
        pltpu.VMEM((W,), jnp.int32),
        pltpu.VMEM((W,), jnp.int32),
        pltpu.VMEM((W,), jnp.int32),
        pltpu.VMEM((W,), jnp.int32),
        pltpu.VMEM((2, W, DH), jnp.float32),
        pltpu.VMEM((ZCH, DH), jnp.float32),
        pltpu.VMEM_SHARED((NP2, DH), jnp.float32),
        pltpu.SemaphoreType.DMA,
    ],
    compiler_params=_sc_params,
)
def _edge_pass(rows_hbm, cols_hbm, zs_hbm, zeros_hbm, out_hbm,
               rowbuf, colbuf, srcbuf, dstbuf, val2, iobuf, acc, g0):
    c = lax.axis_index("c")
    s = lax.axis_index("s")
    iota = lax.iota(jnp.int32, 16)
    pltpu.sync_copy(zeros_hbm.at[pl.ds(0, ZCH)], iobuf)
    for k in range(SUB2 // ZCH):
        pltpu.sync_copy(iobuf, acc.at[pl.ds(s * SUB2 + k * ZCH, ZCH)])
    plsc.subcore_barrier()

    def eoff(w):
        return pl.multiple_of(s * SLICE + w * W, 8)

    def load_idx(w):
        pltpu.sync_copy(cols_hbm.at[pl.ds(eoff(w), W)], colbuf)
        pltpu.sync_copy(rows_hbm.at[pl.ds(eoff(w), W)], rowbuf)

    def srcc():
        for j in range(W // 16):
            row16 = rowbuf[pl.ds(j * 16, 16)]
            srcbuf[pl.ds(j * 16, 16)] = row16 + row16 + c

    def dstc():
        for j in range(W // 16):
            col16 = colbuf[pl.ds(j * 16, 16)]
            garb = (GARB2 + (j % 4) * 16) + iota
            dstbuf[pl.ds(j * 16, 16)] = jnp.where(col16 >= 0, col16, garb)

    load_idx(0)
    srcc()
    dstc()
    pltpu.async_copy(zs_hbm.at[srcbuf], val2.at[0], g0)

    def body(w, carry):
        sl = w & 1
        osl = 1 - sl
        pltpu.make_async_copy(zs_hbm.at[srcbuf], val2.at[sl], g0).wait()

        @pl.when(w < NWIN - 1)
        def _():
            load_idx(w + 1)
            srcc()
            pltpu.async_copy(zs_hbm.at[srcbuf], val2.at[osl], g0)

        pltpu.sync_copy(val2.at[sl], acc.at[dstbuf], add=True)

        @pl.when(w < NWIN - 1)
        def _():
            dstc()

        return carry

    lax.fori_loop(0, NWIN, body, 0)
    plsc.subcore_barrier()
    for k in range(SUB2 // ZCH):
        r0 = s * SUB2 + k * ZCH
        pltpu.sync_copy(acc.at[pl.ds(r0, ZCH)], iobuf)
        pltpu.sync_copy(iobuf, out_hbm.at[pl.ds(c * NP2 + r0, ZCH)])


def _matTdot(x, w):
    return lax.dot_general(x, w, (((1,), (1,)), ((), ())),
                           preferred_element_type=jnp.float32)


def _proj(feat, Wm, bm):
    K = feat.shape[1]

    def body(x_ref, w_ref, b_ref, o_ref):
        o_ref[...] = _matTdot(x_ref[...], w_ref[...]) + b_ref[...]

    return pl.pallas_call(
        body,
        grid=(25,),
        in_specs=[pl.BlockSpec((1000, K), lambda i: (i, 0)),
                  pl.BlockSpec((D, K), lambda i: (0, 0)),
                  pl.BlockSpec((1, D), lambda i: (0, 0))],
        out_specs=pl.BlockSpec((1000, D), lambda i: (i, 0)),
        out_shape=jax.ShapeDtypeStruct((NU, D), jnp.float32),
    )(feat, Wm, bm.reshape(1, D))


_RB = 2000


def _first(X0, Wm, deg2):
    def body(x_ref, w_ref, d_ref, o_ref):
        dinv = lax.rsqrt(d_ref[...] + 1.0)
        o_ref[...] = _matTdot(x_ref[...], w_ref[...]) * dinv

    return pl.pallas_call(
        body,
        grid=(N // _RB,),
        in_specs=[pl.BlockSpec((_RB, D), lambda i: (i, 0)),
                  pl.BlockSpec((D, D), lambda i: (0, 0)),
                  pl.BlockSpec((_RB, 1), lambda i: (i, 0))],
        out_specs=pl.BlockSpec((_RB, D), lambda i: (i, 0)),
        out_shape=jax.ShapeDtypeStruct((N, D), jnp.float32),
    )(X0, Wm, deg2)


def _mid(Se, Zs, deg2, bm, Wm):
    def body(se_ref, zs_ref, d_ref, b_ref, w_ref, o_ref):
        dinv = lax.rsqrt(d_ref[...] + 1.0)
        x = dinv * (se_ref[...] + zs_ref[...]) + b_ref[...]
        o_ref[...] = _matTdot(x, w_ref[...]) * dinv

    return pl.pallas_call(
        body,
        grid=(N // _RB,),
        in_specs=[pl.BlockSpec((_RB, D), lambda i: (i, 0)),
                  pl.BlockSpec((_RB, D), lambda i: (i, 0)),
                  pl.BlockSpec((_RB, 1), lambda i: (i, 0)),
                  pl.BlockSpec((1, D), lambda i: (0, 0)),
                  pl.BlockSpec((D, D), lambda i: (0, 0))],
        out_specs=pl.BlockSpec((_RB, D), lambda i: (i, 0)),
        out_shape=jax.ShapeDtypeStruct((N, D), jnp.float32),
    )(Se, Zs, deg2, bm.reshape(1, D), Wm)


def _fin(X0, Se1, Zs1, Se2, Zs2, deg2, b1m, b2m):
    def body(x0_ref, se1_ref, zs1_ref, se2_ref, zs2_ref, d_ref, b1_ref,
             b2_ref, o_ref):
        dinv = lax.rsqrt(d_ref[...] + 1.0)
        x1 = dinv * (se1_ref[...] + zs1_ref[...]) + b1_ref[...]
        x2 = dinv * (se2_ref[...] + zs2_ref[...]) + b2_ref[...]
        o_ref[...] = (x0_ref[...] + x1 + x2) * (1.0 / 3.0)

    rb = pl.BlockSpec((_RB, D), lambda i: (i, 0))
    return pl.pallas_call(
        body,
        grid=(N // _RB,),
        in_specs=[rb, rb, rb, rb, rb,
                  pl.BlockSpec((_RB, 1), lambda i: (i, 0)),
                  pl.BlockSpec((1, D), lambda i: (0, 0)),
                  pl.BlockSpec((1, D), lambda i: (0, 0))],
        out_specs=rb,
        out_shape=jax.ShapeDtypeStruct((N, D), jnp.float32),
    )(X0, Se1, Zs1, Se2, Zs2, deg2, b1m.reshape(1, D), b2m.reshape(1, D))


def _unpad(a2):
    return jnp.concatenate([a2[:HALF], a2[NP:NP + HALF]], axis=0)


def _unsplit(se):
    return jnp.concatenate([se[:N], se[NP2:NP2 + N]], axis=1)


def kernel(edge_index, v_feat, t_feat, user_emb, item_emb, user_emb_v,
           user_emb_t, Wv, bv, Wt, bt, W1, b1, W2, b2):
    ei = edge_index.astype(jnp.int32)
    src = jnp.concatenate([ei[:, 0], ei[:, 1]])
    dst = jnp.concatenate([ei[:, 1], ei[:, 0]])
    npad = EPT - E2
    src_p = jnp.concatenate([src, jnp.arange(npad, dtype=jnp.int32) % N])
    dst_p = jnp.concatenate([dst, jnp.full((npad,), -1, jnp.int32)])

    zeros1 = jnp.zeros((SUBROWS,), jnp.float32)
    zeros2 = jnp.zeros((ZCH, DH), jnp.float32)

    deg_p = _deg_pass(dst_p, zeros1)
    deg2 = _unpad(deg_p).reshape(N, 1)

    v_emb = _proj(v_feat, Wv, bv)
    t_emb = _proj(t_feat, Wt, bt)

    def propagate(X0):
        Zs1 = _first(X0, W1, deg2)
        Se1 = _unsplit(_edge_pass(src_p, dst_p, Zs1.reshape(2 * N, DH),
                                  zeros2))
        Zs2 = _mid(Se1, Zs1, deg2, b1, W2)
        Se2 = _unsplit(_edge_pass(src_p, dst_p, Zs2.reshape(2 * N, DH),
                                  zeros2))
        M = _fin(X0, Se1, Zs1, Se2, Zs2, deg2, b1, b2)
        return M[:NU], M[NU:]

    u_g, i_g = propagate(jnp.concatenate([user_emb, item_emb], axis=0))
    u_v, i_v = propagate(jnp.concatenate([user_emb_v, v_emb], axis=0))
    u_t, i_t = propagate(jnp.concatenate([user_emb_t, t_emb], axis=0))
    return (u_g, i_g, u_v, i_v, u_t, i_t)


# speedup vs baseline: 2.4685x; 1.6919x over previous
import functools

import jax
import jax.numpy as jnp
from jax import lax
from jax.experimental import pallas as pl
from jax.experimental.pallas import tpu as pltpu
from jax.experimental.pallas import tpu_sc as plsc

NU = 25000
NI = 25000
N = NU + NI
D = 64
E2 = 800000
EPT = 819200
W = 128
NWIN = 400
CW = 8
CHW = CW * W
NCH = NWIN // CW
SLICE = NWIN * W
NP = 25088
HALF = 25000
GARB = 25024
SUBROWS = 1568
NP2 = 50176
GARB2 = 50000
SUB2 = 3136
ZCH = 448
DH = 32

_mesh = plsc.VectorSubcoreMesh(core_axis_name="c", subcore_axis_name="s")
_sc_params = pltpu.CompilerParams(use_tc_tiling_on_sc=False)


@functools.partial(
    pl.kernel,
    out_type=jax.ShapeDtypeStruct((2 * NP,), jnp.float32),
    mesh=_mesh,
    scratch_types=[
        pltpu.VMEM((W,), jnp.int32),
        pltpu.VMEM((W,), jnp.int32),
        pltpu.VMEM((W,), jnp.float32),
        pltpu.VMEM((SUBROWS,), jnp.float32),
        pltpu.VMEM_SHARED((NP,), jnp.float32),
    ],
    compiler_params=_sc_params,
)
def _deg_pass(cols_hbm, zeros_hbm, out_hbm, colbuf, dstbuf, onesbuf, stage,
              acc):
    c = lax.axis_index("c")
    s = lax.axis_index("s")
    base = c * HALF
    iota = lax.iota(jnp.int32, 16)
    one16 = jnp.full((16,), 1.0, jnp.float32)
    for j in range(W // 16):
        onesbuf[pl.ds(j * 16, 16)] = one16
    pltpu.sync_copy(zeros_hbm.at[pl.ds(0, SUBROWS)], stage)
    pltpu.sync_copy(stage, acc.at[pl.ds(s * SUBROWS, SUBROWS)])
    plsc.subcore_barrier()

    def win(w, carry):
        eoff = pl.multiple_of(s * SLICE + w * W, 8)
        pltpu.sync_copy(cols_hbm.at[pl.ds(eoff, W)], colbuf)
        for j in range(W // 16):
            col16 = colbuf[pl.ds(j * 16, 16)]
            local = col16 - base
            inb = (local >= 0) & (local < HALF)
            garb = (GARB + (j % 4) * 16) + iota
            dstbuf[pl.ds(j * 16, 16)] = jnp.where(inb, local, garb)
        pltpu.sync_copy(onesbuf, acc.at[dstbuf], add=True)
        return carry

    lax.fori_loop(0, NWIN, win, 0)
    plsc.subcore_barrier()
    pltpu.sync_copy(acc.at[pl.ds(s * SUBROWS, SUBROWS)], stage)
    pltpu.sync_copy(stage, out_hbm.at[pl.ds(c * NP + s * SUBROWS, SUBROWS)])


@functools.partial(
    pl.kernel,
    out_type=jax.ShapeDtypeStruct((2 * NP2, DH), jnp.float32),
    mesh=_mesh,
    scratch_types=[
        pltpu.VMEM((2 * CHW,), jnp.int32),
        pltpu.VMEM((2 * CHW,), jnp.int32),
        pltpu.VMEM((W,), jnp.int32),
        pltpu.VMEM((W,), jnp.int32),
        pltpu.VMEM((2, W, DH), jnp.float32),
        pltpu.VMEM((ZCH, DH), jnp.float32),
        pltpu.VMEM_SHARED((NP2, DH), jnp.float32),
        pltpu.SemaphoreType.DMA,
        pltpu.SemaphoreType.DMA,
        pltpu.SemaphoreType.DMA,
    ],
    compiler_params=_sc_params,
)
def _edge_pass(rows_hbm, cols_hbm, zs_hbm, zeros_hbm, out_hbm,
               rowch, colch, srcbuf, dstbuf, val2, iobuf, acc, g0, i0, i1):
    c = lax.axis_index("c")
    s = lax.axis_index("s")
    iota = lax.iota(jnp.int32, 16)
    pltpu.sync_copy(zeros_hbm.at[pl.ds(0, ZCH)], iobuf)
    for k in range(SUB2 // ZCH):
        pltpu.sync_copy(iobuf, acc.at[pl.ds(s * SUB2 + k * ZCH, ZCH)])
    plsc.subcore_barrier()

    def choff(k):
        return pl.multiple_of(s * SLICE + k * CHW, 8)

    def issue_chunk(k, slot):
        dst = pl.ds(slot * CHW, CHW)
        pltpu.async_copy(rows_hbm.at[pl.ds(choff(k), CHW)], rowch.at[dst], i0)
        pltpu.async_copy(cols_hbm.at[pl.ds(choff(k), CHW)], colch.at[dst], i1)

    def wait_chunk(k, slot):
        dst = pl.ds(slot * CHW, CHW)
        pltpu.make_async_copy(rows_hbm.at[pl.ds(choff(k), CHW)], rowch.at[dst],
                              i0).wait()
        pltpu.make_async_copy(cols_hbm.at[pl.ds(choff(k), CHW)], colch.at[dst],
                              i1).wait()

    def srcc(slot, lw):
        base = slot * CHW + lw * W
        for j in range(W // 16):
            row16 = rowch[pl.ds(pl.multiple_of(base + j * 16, 16), 16)]
            srcbuf[pl.ds(j * 16, 16)] = row16 + row16 + c

    def dstc(slot, lw):
        base = slot * CHW + lw * W
        for j in range(W // 16):
            col16 = colch[pl.ds(pl.multiple_of(base + j * 16, 16), 16)]
            garb = (GARB2 + (j % 4) * 16) + iota
            dstbuf[pl.ds(j * 16, 16)] = jnp.where(col16 >= 0, col16, garb)

    issue_chunk(0, 0)
    wait_chunk(0, 0)
    issue_chunk(1, 1)
    srcc(0, 0)
    dstc(0, 0)
    pltpu.async_copy(zs_hbm.at[srcbuf], val2.at[0], g0)

    def body(w, carry):
        sl = w & 1
        osl = 1 - sl
        nw = w + 1
        nk = nw >> 3
        nslot = nk & 1
        nlw = nw & 7
        pltpu.make_async_copy(zs_hbm.at[srcbuf], val2.at[sl], g0).wait()

        @pl.when(w < NWIN - 1)
        def _():
            @pl.when(nlw == 0)
            def _():
                wait_chunk(nk, nslot)

            srcc(nslot, nlw)
            pltpu.async_copy(zs_hbm.at[srcbuf], val2.at[osl], g0)

        pltpu.sync_copy(val2.at[sl], acc.at[dstbuf], add=True)

        @pl.when(w < NWIN - 1)
        def _():
            dstc(nslot, nlw)

            @pl.when((nlw == 0) & (nk < NCH - 1))
            def _():
                issue_chunk(nk + 1, 1 - nslot)

        return carry

    lax.fori_loop(0, NWIN, body, 0)
    plsc.subcore_barrier()
    for k in range(SUB2 // ZCH):
        r0 = s * SUB2 + k * ZCH
        pltpu.sync_copy(acc.at[pl.ds(r0, ZCH)], iobuf)
        pltpu.sync_copy(iobuf, out_hbm.at[pl.ds(c * NP2 + r0, ZCH)])


def _matTdot(x, w):
    return lax.dot_general(x, w, (((1,), (1,)), ((), ())),
                           preferred_element_type=jnp.float32)


def _proj(feat, Wm, bm):
    K = feat.shape[1]

    def body(x_ref, w_ref, b_ref, o_ref):
        o_ref[...] = _matTdot(x_ref[...], w_ref[...]) + b_ref[...]

    return pl.pallas_call(
        body,
        grid=(25,),
        in_specs=[pl.BlockSpec((1000, K), lambda i: (i, 0)),
                  pl.BlockSpec((D, K), lambda i: (0, 0)),
                  pl.BlockSpec((1, D), lambda i: (0, 0))],
        out_specs=pl.BlockSpec((1000, D), lambda i: (i, 0)),
        out_shape=jax.ShapeDtypeStruct((NU, D), jnp.float32),
    )(feat, Wm, bm.reshape(1, D))


_RB = 2000


def _first(X0, Wm, deg2):
    def body(x_ref, w_ref, d_ref, o_ref):
        dinv = lax.rsqrt(d_ref[...] + 1.0)
        o_ref[...] = _matTdot(x_ref[...], w_ref[...]) * dinv

    return pl.pallas_call(
        body,
        grid=(N // _RB,),
        in_specs=[pl.BlockSpec((_RB, D), lambda i: (i, 0)),
                  pl.BlockSpec((D, D), lambda i: (0, 0)),
                  pl.BlockSpec((_RB, 1), lambda i: (i, 0))],
        out_specs=pl.BlockSpec((_RB, D), lambda i: (i, 0)),
        out_shape=jax.ShapeDtypeStruct((N, D), jnp.float32),
    )(X0, Wm, deg2)


def _mid(Se, Zs, deg2, bm, Wm):
    def body(se_ref, zs_ref, d_ref, b_ref, w_ref, o_ref):
        dinv = lax.rsqrt(d_ref[...] + 1.0)
        x = dinv * (se_ref[...] + zs_ref[...]) + b_ref[...]
        o_ref[...] = _matTdot(x, w_ref[...]) * dinv

    return pl.pallas_call(
        body,
        grid=(N // _RB,),
        in_specs=[pl.BlockSpec((_RB, D), lambda i: (i, 0)),
                  pl.BlockSpec((_RB, D), lambda i: (i, 0)),
                  pl.BlockSpec((_RB, 1), lambda i: (i, 0)),
                  pl.BlockSpec((1, D), lambda i: (0, 0)),
                  pl.BlockSpec((D, D), lambda i: (0, 0))],
        out_specs=pl.BlockSpec((_RB, D), lambda i: (i, 0)),
        out_shape=jax.ShapeDtypeStruct((N, D), jnp.float32),
    )(Se, Zs, deg2, bm.reshape(1, D), Wm)


def _fin(X0, Se1, Zs1, Se2, Zs2, deg2, b1m, b2m):
    def body(x0_ref, se1_ref, zs1_ref, se2_ref, zs2_ref, d_ref, b1_ref,
             b2_ref, o_ref):
        dinv = lax.rsqrt(d_ref[...] + 1.0)
        x1 = dinv * (se1_ref[...] + zs1_ref[...]) + b1_ref[...]
        x2 = dinv * (se2_ref[...] + zs2_ref[...]) + b2_ref[...]
        o_ref[...] = (x0_ref[...] + x1 + x2) * (1.0 / 3.0)

    rb = pl.BlockSpec((_RB, D), lambda i: (i, 0))
    return pl.pallas_call(
        body,
        grid=(N // _RB,),
        in_specs=[rb, rb, rb, rb, rb,
                  pl.BlockSpec((_RB, 1), lambda i: (i, 0)),
                  pl.BlockSpec((1, D), lambda i: (0, 0)),
                  pl.BlockSpec((1, D), lambda i: (0, 0))],
        out_specs=rb,
        out_shape=jax.ShapeDtypeStruct((N, D), jnp.float32),
    )(X0, Se1, Zs1, Se2, Zs2, deg2, b1m.reshape(1, D), b2m.reshape(1, D))


def _unpad(a2):
    return jnp.concatenate([a2[:HALF], a2[NP:NP + HALF]], axis=0)


def _unsplit(se):
    return jnp.concatenate([se[:N], se[NP2:NP2 + N]], axis=1)


def kernel(edge_index, v_feat, t_feat, user_emb, item_emb, user_emb_v,
           user_emb_t, Wv, bv, Wt, bt, W1, b1, W2, b2):
    ei = edge_index.astype(jnp.int32)
    src = jnp.concatenate([ei[:, 0], ei[:, 1]])
    dst = jnp.concatenate([ei[:, 1], ei[:, 0]])
    npad = EPT - E2
    src_p = jnp.concatenate([src, jnp.arange(npad, dtype=jnp.int32) % N])
    dst_p = jnp.concatenate([dst, jnp.full((npad,), -1, jnp.int32)])

    zeros1 = jnp.zeros((SUBROWS,), jnp.float32)
    zeros2 = jnp.zeros((ZCH, DH), jnp.float32)

    deg_p = _deg_pass(dst_p, zeros1)
    deg2 = _unpad(deg_p).reshape(N, 1)

    v_emb = _proj(v_feat, Wv, bv)
    t_emb = _proj(t_feat, Wt, bt)

    def propagate(X0):
        Zs1 = _first(X0, W1, deg2)
        Se1 = _unsplit(_edge_pass(src_p, dst_p, Zs1.reshape(2 * N, DH),
                                  zeros2))
        Zs2 = _mid(Se1, Zs1, deg2, b1, W2)
        Se2 = _unsplit(_edge_pass(src_p, dst_p, Zs2.reshape(2 * N, DH),
                                  zeros2))
        M = _fin(X0, Se1, Zs1, Se2, Zs2, deg2, b1, b2)
        return M[:NU], M[NU:]

    u_g, i_g = propagate(jnp.concatenate([user_emb, item_emb], axis=0))
    u_v, i_v = propagate(jnp.concatenate([user_emb_v, v_emb], axis=0))
    u_t, i_t = propagate(jnp.concatenate([user_emb_t, t_emb], axis=0))
    return (u_g, i_g, u_v, i_v, u_t, i_t)


# speedup vs baseline: 2.6226x; 1.0624x over previous
import functools

import jax
import jax.numpy as jnp
from jax import lax
from jax.experimental import pallas as pl
from jax.experimental.pallas import tpu as pltpu
from jax.experimental.pallas import tpu_sc as plsc

NU = 25000
NI = 25000
N = NU + NI
D = 64
E2 = 800000
EPT = 819200
W = 128
NWIN = 400
CW = 8
CHW = CW * W
NCH = NWIN // CW
SLICE = NWIN * W
NP = 25088
HALF = 25000
GARB = 25024
SUBROWS = 1568
NP2 = 50176
GARB2 = 50000
SUB2 = 3136
ZCH = 448
DH = 32

_mesh = plsc.VectorSubcoreMesh(core_axis_name="c", subcore_axis_name="s")
_sc_params = pltpu.CompilerParams(use_tc_tiling_on_sc=False)


@functools.partial(
    pl.kernel,
    out_type=jax.ShapeDtypeStruct((2 * NP,), jnp.float32),
    mesh=_mesh,
    scratch_types=[
        pltpu.VMEM((2 * CHW,), jnp.int32),
        pltpu.VMEM((W,), jnp.int32),
        pltpu.VMEM((W,), jnp.float32),
        pltpu.VMEM((SUBROWS,), jnp.float32),
        pltpu.VMEM_SHARED((NP,), jnp.float32),
        pltpu.SemaphoreType.DMA,
    ],
    compiler_params=_sc_params,
)
def _deg_pass(cols_hbm, zeros_hbm, out_hbm, colch, dstbuf, onesbuf, stage,
              acc, i1):
    c = lax.axis_index("c")
    s = lax.axis_index("s")
    base = c * HALF
    iota = lax.iota(jnp.int32, 16)
    one16 = jnp.full((16,), 1.0, jnp.float32)
    for j in range(W // 16):
        onesbuf[pl.ds(j * 16, 16)] = one16
    pltpu.sync_copy(zeros_hbm.at[pl.ds(0, SUBROWS)], stage)
    pltpu.sync_copy(stage, acc.at[pl.ds(s * SUBROWS, SUBROWS)])
    plsc.subcore_barrier()

    def choff(k):
        return pl.multiple_of(s * SLICE + k * CHW, 8)

    def issue_chunk(k, slot):
        pltpu.async_copy(cols_hbm.at[pl.ds(choff(k), CHW)],
                         colch.at[pl.ds(slot * CHW, CHW)], i1)

    def wait_chunk(k, slot):
        pltpu.make_async_copy(cols_hbm.at[pl.ds(choff(k), CHW)],
                              colch.at[pl.ds(slot * CHW, CHW)], i1).wait()

    issue_chunk(0, 0)
    wait_chunk(0, 0)
    issue_chunk(1, 1)

    def win(w, carry):
        k = w >> 3
        slot = k & 1
        lw = w & 7

        @pl.when(((w & 7) == 0) & (w > 0))
        def _():
            wait_chunk(k, slot)

            @pl.when(k < NCH - 1)
            def _():
                issue_chunk(k + 1, 1 - slot)

        cbase = slot * CHW + lw * W
        for j in range(W // 16):
            col16 = colch[pl.ds(pl.multiple_of(cbase + j * 16, 16), 16)]
            local = col16 - base
            inb = (local >= 0) & (local < HALF)
            garb = (GARB + (j % 4) * 16) + iota
            dstbuf[pl.ds(j * 16, 16)] = jnp.where(inb, local, garb)
        pltpu.sync_copy(onesbuf, acc.at[dstbuf], add=True)
        return carry

    lax.fori_loop(0, NWIN, win, 0)
    plsc.subcore_barrier()
    pltpu.sync_copy(acc.at[pl.ds(s * SUBROWS, SUBROWS)], stage)
    pltpu.sync_copy(stage, out_hbm.at[pl.ds(c * NP + s * SUBROWS, SUBROWS)])


@functools.partial(
    pl.kernel,
    out_type=jax.ShapeDtypeStruct((2 * NP2, DH), jnp.float32),
    mesh=_mesh,
    scratch_types=[
        pltpu.VMEM((2 * CHW,), jnp.int32),
        pltpu.VMEM((2 * CHW,), jnp.int32),
        pltpu.VMEM((W,), jnp.int32),
        pltpu.VMEM((W,), jnp.int32),
        pltpu.VMEM((2, W, DH), jnp.float32),
        pltpu.VMEM((ZCH, DH), jnp.float32),
        pltpu.VMEM_SHARED((NP2, DH), jnp.float32),
        pltpu.SemaphoreType.DMA,
        pltpu.SemaphoreType.DMA,
        pltpu.SemaphoreType.DMA,
    ],
    compiler_params=_sc_params,
)
def _edge_pass(rows_hbm, cols_hbm, zs_hbm, zeros_hbm, out_hbm,
               rowch, colch, srcbuf, dstbuf, val2, iobuf, acc, g0, i0, i1):
    c = lax.axis_index("c")
    s = lax.axis_index("s")
    iota = lax.iota(jnp.int32, 16)
    pltpu.sync_copy(zeros_hbm.at[pl.ds(0, ZCH)], iobuf)
    for k in range(SUB2 // ZCH):
        pltpu.sync_copy(iobuf, acc.at[pl.ds(s * SUB2 + k * ZCH, ZCH)])
    plsc.subcore_barrier()

    def choff(k):
        return pl.multiple_of(s * SLICE + k * CHW, 8)

    def issue_chunk(k, slot):
        dst = pl.ds(slot * CHW, CHW)
        pltpu.async_copy(rows_hbm.at[pl.ds(choff(k), CHW)], rowch.at[dst], i0)
        pltpu.async_copy(cols_hbm.at[pl.ds(choff(k), CHW)], colch.at[dst], i1)

    def wait_chunk(k, slot):
        dst = pl.ds(slot * CHW, CHW)
        pltpu.make_async_copy(rows_hbm.at[pl.ds(choff(k), CHW)], rowch.at[dst],
                              i0).wait()
        pltpu.make_async_copy(cols_hbm.at[pl.ds(choff(k), CHW)], colch.at[dst],
                              i1).wait()

    def srcc(slot, lw):
        base = slot * CHW + lw * W
        for j in range(W // 16):
            row16 = rowch[pl.ds(pl.multiple_of(base + j * 16, 16), 16)]
            srcbuf[pl.ds(j * 16, 16)] = row16 + row16 + c

    def dstc(slot, lw):
        base = slot * CHW + lw * W
        for j in range(W // 16):
            col16 = colch[pl.ds(pl.multiple_of(base + j * 16, 16), 16)]
            garb = (GARB2 + (j % 4) * 16) + iota
            dstbuf[pl.ds(j * 16, 16)] = jnp.where(col16 >= 0, col16, garb)

    issue_chunk(0, 0)
    wait_chunk(0, 0)
    issue_chunk(1, 1)
    srcc(0, 0)
    dstc(0, 0)
    pltpu.async_copy(zs_hbm.at[srcbuf], val2.at[0], g0)

    def body(w, carry):
        sl = w & 1
        osl = 1 - sl
        nw = w + 1
        nk = nw >> 3
        nslot = nk & 1
        nlw = nw & 7
        pltpu.make_async_copy(zs_hbm.at[srcbuf], val2.at[sl], g0).wait()

        @pl.when(w < NWIN - 1)
        def _():
            @pl.when(nlw == 0)
            def _():
                wait_chunk(nk, nslot)

            srcc(nslot, nlw)
            pltpu.async_copy(zs_hbm.at[srcbuf], val2.at[osl], g0)

        pltpu.sync_copy(val2.at[sl], acc.at[dstbuf], add=True)

        @pl.when(w < NWIN - 1)
        def _():
            dstc(nslot, nlw)

            @pl.when((nlw == 0) & (nk < NCH - 1))
            def _():
                issue_chunk(nk + 1, 1 - nslot)

        return carry

    lax.fori_loop(0, NWIN, body, 0)
    plsc.subcore_barrier()
    for k in range(SUB2 // ZCH):
        r0 = s * SUB2 + k * ZCH
        pltpu.sync_copy(acc.at[pl.ds(r0, ZCH)], iobuf)
        pltpu.sync_copy(iobuf, out_hbm.at[pl.ds(c * NP2 + r0, ZCH)])


def _matTdot(x, w):
    return lax.dot_general(x, w, (((1,), (1,)), ((), ())),
                           preferred_element_type=jnp.float32)


def _proj(feat, Wm, bm):
    K = feat.shape[1]

    def body(x_ref, w_ref, b_ref, o_ref):
        o_ref[...] = _matTdot(x_ref[...], w_ref[...]) + b_ref[...]

    return pl.pallas_call(
        body,
        grid=(25,),
        in_specs=[pl.BlockSpec((1000, K), lambda i: (i, 0)),
                  pl.BlockSpec((D, K), lambda i: (0, 0)),
                  pl.BlockSpec((1, D), lambda i: (0, 0))],
        out_specs=pl.BlockSpec((1000, D), lambda i: (i, 0)),
        out_shape=jax.ShapeDtypeStruct((NU, D), jnp.float32),
    )(feat, Wm, bm.reshape(1, D))


_RB = 2000


def _first(X0, Wm, deg2):
    def body(x_ref, w_ref, d_ref, o_ref):
        dinv = lax.rsqrt(d_ref[...] + 1.0)
        o_ref[...] = _matTdot(x_ref[...], w_ref[...]) * dinv

    return pl.pallas_call(
        body,
        grid=(N // _RB,),
        in_specs=[pl.BlockSpec((_RB, D), lambda i: (i, 0)),
                  pl.BlockSpec((D, D), lambda i: (0, 0)),
                  pl.BlockSpec((_RB, 1), lambda i: (i, 0))],
        out_specs=pl.BlockSpec((_RB, D), lambda i: (i, 0)),
        out_shape=jax.ShapeDtypeStruct((N, D), jnp.float32),
    )(X0, Wm, deg2)


def _mid(Se, Zs, deg2, bm, Wm):
    def body(se_ref, zs_ref, d_ref, b_ref, w_ref, o_ref):
        dinv = lax.rsqrt(d_ref[...] + 1.0)
        x = dinv * (se_ref[...] + zs_ref[...]) + b_ref[...]
        o_ref[...] = _matTdot(x, w_ref[...]) * dinv

    return pl.pallas_call(
        body,
        grid=(N // _RB,),
        in_specs=[pl.BlockSpec((_RB, D), lambda i: (i, 0)),
                  pl.BlockSpec((_RB, D), lambda i: (i, 0)),
                  pl.BlockSpec((_RB, 1), lambda i: (i, 0)),
                  pl.BlockSpec((1, D), lambda i: (0, 0)),
                  pl.BlockSpec((D, D), lambda i: (0, 0))],
        out_specs=pl.BlockSpec((_RB, D), lambda i: (i, 0)),
        out_shape=jax.ShapeDtypeStruct((N, D), jnp.float32),
    )(Se, Zs, deg2, bm.reshape(1, D), Wm)


def _fin(X0, Se1, Zs1, Se2, Zs2, deg2, b1m, b2m):
    def body(x0_ref, se1_ref, zs1_ref, se2_ref, zs2_ref, d_ref, b1_ref,
             b2_ref, o_ref):
        dinv = lax.rsqrt(d_ref[...] + 1.0)
        x1 = dinv * (se1_ref[...] + zs1_ref[...]) + b1_ref[...]
        x2 = dinv * (se2_ref[...] + zs2_ref[...]) + b2_ref[...]
        o_ref[...] = (x0_ref[...] + x1 + x2) * (1.0 / 3.0)

    rb = pl.BlockSpec((_RB, D), lambda i: (i, 0))
    return pl.pallas_call(
        body,
        grid=(N // _RB,),
        in_specs=[rb, rb, rb, rb, rb,
                  pl.BlockSpec((_RB, 1), lambda i: (i, 0)),
                  pl.BlockSpec((1, D), lambda i: (0, 0)),
                  pl.BlockSpec((1, D), lambda i: (0, 0))],
        out_specs=rb,
        out_shape=jax.ShapeDtypeStruct((N, D), jnp.float32),
    )(X0, Se1, Zs1, Se2, Zs2, deg2, b1m.reshape(1, D), b2m.reshape(1, D))


def _unpad(a2):
    return jnp.concatenate([a2[:HALF], a2[NP:NP + HALF]], axis=0)


def _unsplit(se):
    return jnp.concatenate([se[:N], se[NP2:NP2 + N]], axis=1)


def kernel(edge_index, v_feat, t_feat, user_emb, item_emb, user_emb_v,
           user_emb_t, Wv, bv, Wt, bt, W1, b1, W2, b2):
    ei = edge_index.astype(jnp.int32)
    src = jnp.concatenate([ei[:, 0], ei[:, 1]])
    dst = jnp.concatenate([ei[:, 1], ei[:, 0]])
    npad = EPT - E2
    src_p = jnp.concatenate([src, jnp.arange(npad, dtype=jnp.int32) % N])
    dst_p = jnp.concatenate([dst, jnp.full((npad,), -1, jnp.int32)])

    zeros1 = jnp.zeros((SUBROWS,), jnp.float32)
    zeros2 = jnp.zeros((ZCH, DH), jnp.float32)

    deg_p = _deg_pass(dst_p, zeros1)
    deg2 = _unpad(deg_p).reshape(N, 1)

    v_emb = _proj(v_feat, Wv, bv)
    t_emb = _proj(t_feat, Wt, bt)

    def propagate(X0):
        Zs1 = _first(X0, W1, deg2)
        Se1 = _unsplit(_edge_pass(src_p, dst_p, Zs1.reshape(2 * N, DH),
                                  zeros2))
        Zs2 = _mid(Se1, Zs1, deg2, b1, W2)
        Se2 = _unsplit(_edge_pass(src_p, dst_p, Zs2.reshape(2 * N, DH),
                                  zeros2))
        M = _fin(X0, Se1, Zs1, Se2, Zs2, deg2, b1, b2)
        return M[:NU], M[NU:]

    u_g, i_g = propagate(jnp.concatenate([user_emb, item_emb], axis=0))
    u_v, i_v = propagate(jnp.concatenate([user_emb_v, v_emb], axis=0))
    u_t, i_t = propagate(jnp.concatenate([user_emb_t, t_emb], axis=0))
    return (u_g, i_g, u_v, i_v, u_t, i_t)
